# Initial kernel scaffold; baseline (speedup 1.0000x reference)
import jax
import jax.numpy as jnp
from jax.experimental import pallas as pl


def kernel(node_feat, edge_index, w_n2l, b_n2l, p_conv, w1, b1, w2, b2):
    raise NotImplementedError("write your pallas kernel here")


# speedup vs baseline: 6.8998x; 6.8998x over previous
import functools

import jax
import jax.numpy as jnp
from jax import lax
from jax.experimental import pallas as pl
from jax.experimental.pallas import tpu as pltpu
from jax.experimental.pallas import tpu_sc as plsc

N = 50000
E = 800000
LAT = 64
HALF = 32
HID = 128
MAX_LV = 3

NC = 2
NS = 16
EPT = E // NS
CHUNK = 400
NCHUNK = EPT // CHUNK
NPAD = 50176
RPT = NPAD // NS
ZR = 392
NZ = RPT // ZR

ROWB = 1000
NB = N // ROWB


def _sc_segment_sum():
    mesh = plsc.VectorSubcoreMesh(core_axis_name="c", subcore_axis_name="s")

    @functools.partial(
        pl.kernel,
        out_type=jax.ShapeDtypeStruct((NC, NPAD, HALF), jnp.float32),
        mesh=mesh,
        compiler_params=pltpu.CompilerParams(use_tc_tiling_on_sc=False),
        scratch_types=[
            pltpu.VMEM((CHUNK,), jnp.int32),
            pltpu.VMEM((CHUNK,), jnp.int32),
            pltpu.VMEM((CHUNK, HALF), jnp.float32),
            pltpu.VMEM((ZR, HALF), jnp.float32),
            pltpu.VMEM_SHARED((NPAD, HALF), jnp.float32),
            pltpu.SemaphoreType.DMA,
        ],
    )
    def seg(cur_hbm, src_hbm, dst_hbm, out_hbm,
            src_v, dst_v, rows_v, zero_v, acc_sh, sem):
        c = lax.axis_index("c")
        s = lax.axis_index("s")

        @pl.loop(0, ZR)
        def _(i):
            zero_v[i, pl.ds(0, 16)] = jnp.zeros((16,), jnp.float32)
            zero_v[i, pl.ds(16, 16)] = jnp.zeros((16,), jnp.float32)

        @pl.loop(0, NZ)
        def _(k):
            pltpu.sync_copy(zero_v, acc_sh.at[pl.ds(s * RPT + k * ZR, ZR)])

        plsc.subcore_barrier()

        @pl.loop(0, NCHUNK)
        def _(k):
            off = s * EPT + k * CHUNK
            pltpu.sync_copy(src_hbm.at[pl.ds(off, CHUNK)], src_v)
            pltpu.sync_copy(dst_hbm.at[pl.ds(off, CHUNK)], dst_v)
            pltpu.async_copy(cur_hbm.at[c].at[src_v], rows_v, sem).wait()
            pltpu.sync_copy(rows_v, acc_sh.at[dst_v], add=True)

        plsc.subcore_barrier()
        pltpu.sync_copy(acc_sh.at[pl.ds(s * RPT, RPT)],
                        out_hbm.at[c].at[pl.ds(s * RPT, RPT)])

    return seg


_SC_SEG = _sc_segment_sum()


def _init_body(nf_ref, w_ref, b_ref, msg_ref, pair_ref):
    x = nf_ref[...]
    w = w_ref[...]
    y = jnp.dot(x, w, preferred_element_type=jnp.float32) + b_ref[...]
    y = jnp.maximum(y, 0.0)
    msg_ref[...] = y
    pair_ref[0, :, :] = y[:, :HALF]
    pair_ref[1, :, :] = y[:, HALF:]


_tc_init = pl.pallas_call(
    _init_body,
    grid=(NB,),
    in_specs=[
        pl.BlockSpec((ROWB, 2), lambda i: (i, 0)),
        pl.BlockSpec((2, LAT), lambda i: (0, 0)),
        pl.BlockSpec((1, LAT), lambda i: (0, 0)),
    ],
    out_specs=[
        pl.BlockSpec((ROWB, LAT), lambda i: (i, 0)),
        pl.BlockSpec((NC, ROWB, HALF), lambda i: (0, i, 0)),
    ],
    out_shape=[
        jax.ShapeDtypeStruct((N, LAT), jnp.float32),
        jax.ShapeDtypeStruct((NC, N, HALF), jnp.float32),
    ],
)


def _round_body(pair_ref, msg_ref, pc_ref, out_ref):
    x = jnp.concatenate([pair_ref[0], pair_ref[1]], axis=1)
    y = jnp.dot(x, pc_ref[...], preferred_element_type=jnp.float32)
    y = jnp.maximum(y + msg_ref[...], 0.0)
    out_ref[0, :, :] = y[:, :HALF]
    out_ref[1, :, :] = y[:, HALF:]


_tc_round = pl.pallas_call(
    _round_body,
    grid=(NB,),
    in_specs=[
        pl.BlockSpec((NC, ROWB, HALF), lambda i: (0, i, 0)),
        pl.BlockSpec((ROWB, LAT), lambda i: (i, 0)),
        pl.BlockSpec((LAT, LAT), lambda i: (0, 0)),
    ],
    out_specs=pl.BlockSpec((NC, ROWB, HALF), lambda i: (0, i, 0)),
    out_shape=jax.ShapeDtypeStruct((NC, N, HALF), jnp.float32),
)


def _final_body(pair_ref, msg_ref, pc_ref, w1_ref, b1_ref, w2_ref, b2_ref,
                out_ref):
    x = jnp.concatenate([pair_ref[0], pair_ref[1]], axis=1)
    cur = jnp.dot(x, pc_ref[...], preferred_element_type=jnp.float32)
    cur = jnp.maximum(cur + msg_ref[...], 0.0)
    h = jnp.dot(cur, w1_ref[...], preferred_element_type=jnp.float32)
    h = jnp.maximum(h + b1_ref[...], 0.0)
    out_ref[...] = (jnp.dot(h, w2_ref[...], preferred_element_type=jnp.float32)
                    + b2_ref[...])


_tc_final = pl.pallas_call(
    _final_body,
    grid=(NB,),
    in_specs=[
        pl.BlockSpec((NC, ROWB, HALF), lambda i: (0, i, 0)),
        pl.BlockSpec((ROWB, LAT), lambda i: (i, 0)),
        pl.BlockSpec((LAT, LAT), lambda i: (0, 0)),
        pl.BlockSpec((LAT, HID), lambda i: (0, 0)),
        pl.BlockSpec((1, HID), lambda i: (0, 0)),
        pl.BlockSpec((HID, 1), lambda i: (0, 0)),
        pl.BlockSpec((1, 1), lambda i: (0, 0)),
    ],
    out_specs=pl.BlockSpec((ROWB, 1), lambda i: (i, 0)),
    out_shape=jax.ShapeDtypeStruct((N, 1), jnp.float32),
)


def kernel(node_feat, edge_index, w_n2l, b_n2l, p_conv, w1, b1, w2, b2):
    src = edge_index[0].astype(jnp.int32)
    dst = edge_index[1].astype(jnp.int32)
    b_n2l_r = b_n2l.reshape(1, LAT)
    b1_r = b1.reshape(1, HID)
    b2_r = b2.reshape(1, 1)

    msg, pair = _tc_init(node_feat, w_n2l, b_n2l_r)
    out = None
    for lv in range(MAX_LV):
        pooled = _SC_SEG(pair, src, dst)
        if lv < MAX_LV - 1:
            pair = _tc_round(pooled, msg, p_conv)
        else:
            out = _tc_final(pooled, msg, p_conv, w1, b1_r, w2, b2_r)
    return out


# speedup vs baseline: 7.1365x; 1.0343x over previous
import functools

import jax
import jax.numpy as jnp
from jax import lax
from jax.experimental import pallas as pl
from jax.experimental.pallas import tpu as pltpu
from jax.experimental.pallas import tpu_sc as plsc

N = 50000
E = 800000
LAT = 64
HALF = 32
HID = 128
MAX_LV = 3

NC = 2
NS = 16
EPT = E // NS
CHUNK = 200
NCHUNK = EPT // CHUNK
NPAD = 50176
RPT = NPAD // NS
ZR = 196
NZ = RPT // ZR

ROWB = 1000
NB = N // ROWB


def _sc_segment_sum():
    mesh = plsc.VectorSubcoreMesh(core_axis_name="c", subcore_axis_name="s")

    @functools.partial(
        pl.kernel,
        out_type=jax.ShapeDtypeStruct((NC, NPAD, HALF), jnp.float32),
        mesh=mesh,
        compiler_params=pltpu.CompilerParams(use_tc_tiling_on_sc=False),
        scratch_types=[
            pltpu.VMEM((CHUNK,), jnp.int32),
            pltpu.VMEM((CHUNK,), jnp.int32),
            pltpu.VMEM((CHUNK,), jnp.int32),
            pltpu.VMEM((CHUNK,), jnp.int32),
            pltpu.VMEM((CHUNK, HALF), jnp.float32),
            pltpu.VMEM((CHUNK, HALF), jnp.float32),
            pltpu.VMEM((ZR, HALF), jnp.float32),
            pltpu.VMEM_SHARED((NPAD, HALF), jnp.float32),
            pltpu.SemaphoreType.DMA,
            pltpu.SemaphoreType.DMA,
        ],
    )
    def seg(cur_hbm, src_hbm, dst_hbm, out_hbm,
            src_v0, dst_v0, src_v1, dst_v1, rows_v0, rows_v1,
            zero_v, acc_sh, sem0, sem1):
        c = lax.axis_index("c")
        s = lax.axis_index("s")

        @pl.loop(0, ZR)
        def _(i):
            zero_v[i, pl.ds(0, 16)] = jnp.zeros((16,), jnp.float32)
            zero_v[i, pl.ds(16, 16)] = jnp.zeros((16,), jnp.float32)

        @pl.loop(0, NZ)
        def _(k):
            pltpu.sync_copy(zero_v, acc_sh.at[pl.ds(s * RPT + k * ZR, ZR)])

        plsc.subcore_barrier()

        ebase = s * EPT

        def load_idx(k, sv, dv):
            pltpu.sync_copy(src_hbm.at[pl.ds(ebase + k * CHUNK, CHUNK)], sv)
            pltpu.sync_copy(dst_hbm.at[pl.ds(ebase + k * CHUNK, CHUNK)], dv)

        def gather(sv, rv, sem):
            pltpu.async_copy(cur_hbm.at[c].at[sv], rv, sem)

        def drain_scatter(sv, dv, rv, sem):
            pltpu.make_async_copy(cur_hbm.at[c].at[sv], rv, sem).wait()
            pltpu.sync_copy(rv, acc_sh.at[dv], add=True)

        load_idx(0, src_v0, dst_v0)
        gather(src_v0, rows_v0, sem0)
        load_idx(1, src_v1, dst_v1)
        gather(src_v1, rows_v1, sem1)

        @pl.loop(0, (NCHUNK - 2) // 2)
        def _(k2):
            base = 2 * k2
            drain_scatter(src_v0, dst_v0, rows_v0, sem0)
            load_idx(base + 2, src_v0, dst_v0)
            gather(src_v0, rows_v0, sem0)
            drain_scatter(src_v1, dst_v1, rows_v1, sem1)
            load_idx(base + 3, src_v1, dst_v1)
            gather(src_v1, rows_v1, sem1)

        drain_scatter(src_v0, dst_v0, rows_v0, sem0)
        drain_scatter(src_v1, dst_v1, rows_v1, sem1)

        plsc.subcore_barrier()
        pltpu.sync_copy(acc_sh.at[pl.ds(s * RPT, RPT)],
                        out_hbm.at[c].at[pl.ds(s * RPT, RPT)])

    return seg


_SC_SEG = _sc_segment_sum()


def _init_body(nf_ref, w_ref, b_ref, msg_ref, pair_ref):
    x = nf_ref[...]
    w = w_ref[...]
    y = jnp.dot(x, w, preferred_element_type=jnp.float32) + b_ref[...]
    y = jnp.maximum(y, 0.0)
    msg_ref[...] = y
    pair_ref[0, :, :] = y[:, :HALF]
    pair_ref[1, :, :] = y[:, HALF:]


_tc_init = pl.pallas_call(
    _init_body,
    grid=(NB,),
    in_specs=[
        pl.BlockSpec((ROWB, 2), lambda i: (i, 0)),
        pl.BlockSpec((2, LAT), lambda i: (0, 0)),
        pl.BlockSpec((1, LAT), lambda i: (0, 0)),
    ],
    out_specs=[
        pl.BlockSpec((ROWB, LAT), lambda i: (i, 0)),
        pl.BlockSpec((NC, ROWB, HALF), lambda i: (0, i, 0)),
    ],
    out_shape=[
        jax.ShapeDtypeStruct((N, LAT), jnp.float32),
        jax.ShapeDtypeStruct((NC, N, HALF), jnp.float32),
    ],
)


def _round_body(pair_ref, msg_ref, pc_ref, out_ref):
    x = jnp.concatenate([pair_ref[0], pair_ref[1]], axis=1)
    y = jnp.dot(x, pc_ref[...], preferred_element_type=jnp.float32)
    y = jnp.maximum(y + msg_ref[...], 0.0)
    out_ref[0, :, :] = y[:, :HALF]
    out_ref[1, :, :] = y[:, HALF:]


_tc_round = pl.pallas_call(
    _round_body,
    grid=(NB,),
    in_specs=[
        pl.BlockSpec((NC, ROWB, HALF), lambda i: (0, i, 0)),
        pl.BlockSpec((ROWB, LAT), lambda i: (i, 0)),
        pl.BlockSpec((LAT, LAT), lambda i: (0, 0)),
    ],
    out_specs=pl.BlockSpec((NC, ROWB, HALF), lambda i: (0, i, 0)),
    out_shape=jax.ShapeDtypeStruct((NC, N, HALF), jnp.float32),
)


def _final_body(pair_ref, msg_ref, pc_ref, w1_ref, b1_ref, w2_ref, b2_ref,
                out_ref):
    x = jnp.concatenate([pair_ref[0], pair_ref[1]], axis=1)
    cur = jnp.dot(x, pc_ref[...], preferred_element_type=jnp.float32)
    cur = jnp.maximum(cur + msg_ref[...], 0.0)
    h = jnp.dot(cur, w1_ref[...], preferred_element_type=jnp.float32)
    h = jnp.maximum(h + b1_ref[...], 0.0)
    out_ref[...] = (jnp.dot(h, w2_ref[...], preferred_element_type=jnp.float32)
                    + b2_ref[...])


_tc_final = pl.pallas_call(
    _final_body,
    grid=(NB,),
    in_specs=[
        pl.BlockSpec((NC, ROWB, HALF), lambda i: (0, i, 0)),
        pl.BlockSpec((ROWB, LAT), lambda i: (i, 0)),
        pl.BlockSpec((LAT, LAT), lambda i: (0, 0)),
        pl.BlockSpec((LAT, HID), lambda i: (0, 0)),
        pl.BlockSpec((1, HID), lambda i: (0, 0)),
        pl.BlockSpec((HID, 1), lambda i: (0, 0)),
        pl.BlockSpec((1, 1), lambda i: (0, 0)),
    ],
    out_specs=pl.BlockSpec((ROWB, 1), lambda i: (i, 0)),
    out_shape=jax.ShapeDtypeStruct((N, 1), jnp.float32),
)


def kernel(node_feat, edge_index, w_n2l, b_n2l, p_conv, w1, b1, w2, b2):
    src = edge_index[0].astype(jnp.int32)
    dst = edge_index[1].astype(jnp.int32)
    b_n2l_r = b_n2l.reshape(1, LAT)
    b1_r = b1.reshape(1, HID)
    b2_r = b2.reshape(1, 1)

    msg, pair = _tc_init(node_feat, w_n2l, b_n2l_r)
    out = None
    for lv in range(MAX_LV):
        pooled = _SC_SEG(pair, src, dst)
        if lv < MAX_LV - 1:
            pair = _tc_round(pooled, msg, p_conv)
        else:
            out = _tc_final(pooled, msg, p_conv, w1, b1_r, w2, b2_r)
    return out


# speedup vs baseline: 8.2963x; 1.1625x over previous
import functools

import jax
import jax.numpy as jnp
from jax import lax
from jax.experimental import pallas as pl
from jax.experimental.pallas import tpu as pltpu
from jax.experimental.pallas import tpu_sc as plsc

N = 50000
E = 800000
LAT = 64
HID = 128
MAX_LV = 3

NC = 2
NS = 16
EPT = E // NS

NHALF = N // NC
APAD = 25088
RPT = APAD // NS
DUMMY = 25080
ZR = 112
NZ = RPT // ZR

FB = 2000
NBLK = EPT // FB
FSTG = FB + 16
CHUNK = 128
CAPR = 52224
MAXCH = CAPR // CHUNK

ROWB = 1000
NB = N // ROWB
NBH = NHALF // ROWB

_MESH = plsc.VectorSubcoreMesh(core_axis_name="c", subcore_axis_name="s")
_SC_PARAMS = pltpu.CompilerParams(use_tc_tiling_on_sc=False,
                                  needs_layout_passes=False)
_SC_FILTER_PARAMS = _SC_PARAMS


@functools.partial(
    pl.kernel,
    out_type=[
        jax.ShapeDtypeStruct((NC, NS, CAPR), jnp.int32),
        jax.ShapeDtypeStruct((NC, NS, CAPR), jnp.int32),
        jax.ShapeDtypeStruct((NC, NS, 16), jnp.int32),
    ],
    mesh=_MESH,
    compiler_params=_SC_FILTER_PARAMS,
    scratch_types=[
        pltpu.VMEM((FB,), jnp.int32),
        pltpu.VMEM((FB,), jnp.int32),
        pltpu.VMEM((FSTG,), jnp.int32),
        pltpu.VMEM((FSTG,), jnp.int32),
        pltpu.VMEM((16,), jnp.int32),
    ],
)
def _sc_filter(src_hbm, dst_hbm, fsrc_hbm, fdst_hbm, fcnt_hbm,
               sv, dv, osrc, odst, cntbuf):
    c = lax.axis_index("c")
    s = lax.axis_index("s")
    lo = c * NHALF
    lanes = lax.iota(jnp.int32, 16)
    dummy_d = jnp.full((16,), DUMMY, jnp.int32)
    dummy_s = jnp.zeros((16,), jnp.int32)

    def block(b, off):
        ebase = s * EPT + b * FB
        pltpu.sync_copy(src_hbm.at[pl.ds(ebase, FB)], sv)
        pltpu.sync_copy(dst_hbm.at[pl.ds(ebase, FB)], dv)

        def group(g, cnt_v):
            d = dv[pl.ds(g * 16, 16)]
            sr = sv[pl.ds(g * 16, 16)]
            dl = d - lo
            m = (dl >= 0) & (dl < NHALF)
            mi = m.astype(jnp.int32)
            pos = cnt_v + plsc.cumsum(mi) - 1
            plsc.store_scatter(odst, [pos], dl, mask=m)
            plsc.store_scatter(osrc, [pos], sr, mask=m)
            return cnt_v + plsc.all_reduce_population_count(m)

        cnt_v = lax.fori_loop(0, FB // 16, group, jnp.zeros((16,), jnp.int32))
        plsc.store_scatter(odst, [cnt_v + lanes], dummy_d)
        plsc.store_scatter(osrc, [cnt_v + lanes], dummy_s)
        cnt = lax.reduce_max(cnt_v, (0,))
        cnt8 = jnp.bitwise_and(cnt + 7, -8)
        offa = pl.multiple_of(off, 8)
        pltpu.sync_copy(osrc, fsrc_hbm.at[c].at[s].at[pl.ds(offa, FSTG)])
        pltpu.sync_copy(odst, fdst_hbm.at[c].at[s].at[pl.ds(offa, FSTG)])
        return off + cnt8

    off = lax.fori_loop(0, NBLK, block, jnp.int32(0))

    @pl.loop(0, FSTG // 16)
    def _(i):
        plsc.store_scatter(odst, [i * 16 + lanes], dummy_d)
        plsc.store_scatter(osrc, [i * 16 + lanes], dummy_s)

    offa = pl.multiple_of(off, 8)
    pltpu.sync_copy(osrc, fsrc_hbm.at[c].at[s].at[pl.ds(offa, FSTG)])
    pltpu.sync_copy(odst, fdst_hbm.at[c].at[s].at[pl.ds(offa, FSTG)])

    cntbuf[pl.ds(0, 16)] = jnp.full((16,), 1, jnp.int32) * off
    pltpu.sync_copy(cntbuf, fcnt_hbm.at[c].at[s])


@functools.partial(
    pl.kernel,
    out_type=jax.ShapeDtypeStruct((NC, APAD, LAT), jnp.float32),
    mesh=_MESH,
    compiler_params=_SC_PARAMS,
    scratch_types=[
        pltpu.VMEM((CHUNK,), jnp.int32),
        pltpu.VMEM((CHUNK,), jnp.int32),
        pltpu.VMEM((CHUNK,), jnp.int32),
        pltpu.VMEM((CHUNK,), jnp.int32),
        pltpu.VMEM((CHUNK, LAT), jnp.float32),
        pltpu.VMEM((CHUNK, LAT), jnp.float32),
        pltpu.VMEM((16,), jnp.int32),
        pltpu.VMEM((ZR, LAT), jnp.float32),
        pltpu.VMEM_SHARED((APAD, LAT), jnp.float32),
        pltpu.SemaphoreType.DMA,
        pltpu.SemaphoreType.DMA,
    ],
)
def _sc_round(cur_hbm, fsrc_hbm, fdst_hbm, fcnt_hbm, out_hbm,
              src_v0, dst_v0, src_v1, dst_v1, rows_v0, rows_v1,
              cnt_v, zero_v, acc_sh, sem0, sem1):
    c = lax.axis_index("c")
    s = lax.axis_index("s")

    pltpu.sync_copy(fcnt_hbm.at[c].at[s], cnt_v)
    nedges = lax.reduce_max(cnt_v[pl.ds(0, 16)], (0,))
    nch = (nedges + CHUNK - 1) // CHUNK

    @pl.loop(0, ZR)
    def _(i):
        zero_v[i, pl.ds(0, 16)] = jnp.zeros((16,), jnp.float32)
        zero_v[i, pl.ds(16, 16)] = jnp.zeros((16,), jnp.float32)
        zero_v[i, pl.ds(32, 16)] = jnp.zeros((16,), jnp.float32)
        zero_v[i, pl.ds(48, 16)] = jnp.zeros((16,), jnp.float32)

    @pl.loop(0, NZ)
    def _(k):
        pltpu.sync_copy(zero_v, acc_sh.at[pl.ds(s * RPT + k * ZR, ZR)])

    plsc.subcore_barrier()

    freg_s = fsrc_hbm.at[c].at[s]
    freg_d = fdst_hbm.at[c].at[s]

    def start(k, sv, dv, rv, sem):
        @pl.when(k < nch)
        def _():
            pltpu.sync_copy(freg_s.at[pl.ds(k * CHUNK, CHUNK)], sv)
            pltpu.sync_copy(freg_d.at[pl.ds(k * CHUNK, CHUNK)], dv)
            pltpu.async_copy(cur_hbm.at[sv], rv, sem)

    def drain_scatter(k, sv, dv, rv, sem):
        @pl.when(k < nch)
        def _():
            pltpu.make_async_copy(cur_hbm.at[sv], rv, sem).wait()
            pltpu.sync_copy(rv, acc_sh.at[dv], add=True)

    start(0, src_v0, dst_v0, rows_v0, sem0)
    start(1, src_v1, dst_v1, rows_v1, sem1)

    @pl.loop(0, MAXCH // 2)
    def _(k2):
        base = 2 * k2
        drain_scatter(base, src_v0, dst_v0, rows_v0, sem0)
        start(base + 2, src_v0, dst_v0, rows_v0, sem0)
        drain_scatter(base + 1, src_v1, dst_v1, rows_v1, sem1)
        start(base + 3, src_v1, dst_v1, rows_v1, sem1)

    plsc.subcore_barrier()
    pltpu.sync_copy(acc_sh.at[pl.ds(s * RPT, RPT)],
                    out_hbm.at[c].at[pl.ds(s * RPT, RPT)])


def _init_body(nf_ref, w_ref, b_ref, msg_ref):
    x = nf_ref[...]
    w = w_ref[...]
    y = jnp.dot(x, w, preferred_element_type=jnp.float32) + b_ref[...]
    msg_ref[...] = jnp.maximum(y, 0.0)


_tc_init = pl.pallas_call(
    _init_body,
    grid=(NB,),
    in_specs=[
        pl.BlockSpec((ROWB, 2), lambda i: (i, 0)),
        pl.BlockSpec((2, LAT), lambda i: (0, 0)),
        pl.BlockSpec((1, LAT), lambda i: (0, 0)),
    ],
    out_specs=pl.BlockSpec((ROWB, LAT), lambda i: (i, 0)),
    out_shape=jax.ShapeDtypeStruct((N, LAT), jnp.float32),
)


def _round_body(pool_ref, msg_ref, pc_ref, out_ref):
    x = pool_ref[0]
    y = jnp.dot(x, pc_ref[...], preferred_element_type=jnp.float32)
    out_ref[...] = jnp.maximum(y + msg_ref[...], 0.0)


_POOL_SPEC = pl.BlockSpec((1, ROWB, LAT), lambda i: (i // NBH, i % NBH, 0))

_tc_round = pl.pallas_call(
    _round_body,
    grid=(NB,),
    in_specs=[
        _POOL_SPEC,
        pl.BlockSpec((ROWB, LAT), lambda i: (i, 0)),
        pl.BlockSpec((LAT, LAT), lambda i: (0, 0)),
    ],
    out_specs=pl.BlockSpec((ROWB, LAT), lambda i: (i, 0)),
    out_shape=jax.ShapeDtypeStruct((N, LAT), jnp.float32),
)


def _final_body(pool_ref, msg_ref, pc_ref, w1_ref, b1_ref, w2_ref, b2_ref,
                out_ref):
    x = pool_ref[0]
    cur = jnp.dot(x, pc_ref[...], preferred_element_type=jnp.float32)
    cur = jnp.maximum(cur + msg_ref[...], 0.0)
    h = jnp.dot(cur, w1_ref[...], preferred_element_type=jnp.float32)
    h = jnp.maximum(h + b1_ref[...], 0.0)
    out_ref[...] = (jnp.dot(h, w2_ref[...], preferred_element_type=jnp.float32)
                    + b2_ref[...])


_tc_final = pl.pallas_call(
    _final_body,
    grid=(NB,),
    in_specs=[
        _POOL_SPEC,
        pl.BlockSpec((ROWB, LAT), lambda i: (i, 0)),
        pl.BlockSpec((LAT, LAT), lambda i: (0, 0)),
        pl.BlockSpec((LAT, HID), lambda i: (0, 0)),
        pl.BlockSpec((1, HID), lambda i: (0, 0)),
        pl.BlockSpec((HID, 1), lambda i: (0, 0)),
        pl.BlockSpec((1, 1), lambda i: (0, 0)),
    ],
    out_specs=pl.BlockSpec((ROWB, 1), lambda i: (i, 0)),
    out_shape=jax.ShapeDtypeStruct((N, 1), jnp.float32),
)


def kernel(node_feat, edge_index, w_n2l, b_n2l, p_conv, w1, b1, w2, b2):
    src = edge_index[0].astype(jnp.int32)
    dst = edge_index[1].astype(jnp.int32)
    b_n2l_r = b_n2l.reshape(1, LAT)
    b1_r = b1.reshape(1, HID)
    b2_r = b2.reshape(1, 1)

    fsrc, fdst, fcnt = _sc_filter(src, dst)
    cur = _tc_init(node_feat, w_n2l, b_n2l_r)
    msg = cur
    out = None
    for lv in range(MAX_LV):
        pooled = _sc_round(cur, fsrc, fdst, fcnt)
        if lv < MAX_LV - 1:
            cur = _tc_round(pooled, msg, p_conv)
        else:
            out = _tc_final(pooled, msg, p_conv, w1, b1_r, w2, b2_r)
    return out
